# baseline (speedup 1.0000x reference)
import functools

import jax
import jax.numpy as jnp
from jax import lax
from jax.experimental import pallas as pl
from jax.experimental.pallas import tpu as pltpu
from jax.experimental.pallas import tpu_sc as plsc

EMB = 64
NC = 2
NS = 16
NW = NC * NS
CHUNK = 640


def _lookup(idx, table):
    B = idx.shape[0]
    assert B % NW == 0
    bpw = B // NW
    assert bpw % CHUNK == 0 and (bpw // CHUNK) % 2 == 0
    nchunk = bpw // CHUNK

    mesh = plsc.VectorSubcoreMesh(
        core_axis_name="c", subcore_axis_name="s", num_cores=NC, num_subcores=NS
    )

    @functools.partial(
        pl.kernel,
        out_type=jax.ShapeDtypeStruct((B, EMB), jnp.float32),
        mesh=mesh,
        scratch_types=[
            pltpu.VMEM((bpw,), jnp.int32),
            pltpu.VMEM((CHUNK, EMB), jnp.float32),
            pltpu.VMEM((CHUNK, EMB), jnp.float32),
            pltpu.SemaphoreType.DMA,
            pltpu.SemaphoreType.DMA,
            pltpu.SemaphoreType.DMA,
            pltpu.SemaphoreType.DMA,
        ],
        compiler_params=pltpu.CompilerParams(use_tc_tiling_on_sc=False),
    )
    def body(idx_hbm, table_hbm, out_hbm, idx_v, rows0, rows1, g0, g1, o0, o1):
        wid = lax.axis_index("s") * NC + lax.axis_index("c")
        base = wid * bpw
        rows = (rows0, rows1)
        gsem = (g0, g1)
        osem = (o0, o1)

        pltpu.sync_copy(idx_hbm.at[pl.ds(base, bpw)], idx_v)

        def g_start(t, b):
            pltpu.async_copy(
                table_hbm.at[idx_v.at[pl.ds(t * CHUNK, CHUNK)]], rows[b], gsem[b]
            )

        def g_wait(b):
            pltpu.make_async_copy(
                table_hbm.at[idx_v.at[pl.ds(0, CHUNK)]], rows[b], gsem[b]
            ).wait()

        def o_start(t, b):
            pltpu.async_copy(
                rows[b], out_hbm.at[pl.ds(base + t * CHUNK, CHUNK)], osem[b]
            )

        def o_wait(b):
            pltpu.make_async_copy(
                rows[b], out_hbm.at[pl.ds(base, CHUNK)], osem[b]
            ).wait()

        g_start(0, 0)
        g_start(1, 1)
        g_wait(0)
        o_start(0, 0)

        @pl.loop(1, nchunk - 1, step=2)
        def mid(c):
            for b in (1, 0):
                t = c if b == 1 else c + 1
                nb = 1 - b
                o_wait(nb)
                g_start(t + 1, nb)
                g_wait(b)
                o_start(t, b)

        g_wait(1)
        o_start(nchunk - 1, 1)
        o_wait(0)
        o_wait(1)

    return body(idx, table)


def kernel(x, emb_weight):
    b, h = x.shape
    padded = jnp.pad(emb_weight, ((0, 0), (0, 64))).reshape(2000000, EMB)
    outs = []
    for h0 in (0, h // 2):
        idx = x[:, h0 : h0 + h // 2].reshape(-1).astype(jnp.int32) * 2
        outs.append(_lookup(idx, padded).reshape(b, h // 2, EMB))
    return jnp.concatenate(outs, axis=1)


# speedup vs baseline: 1.0935x; 1.0935x over previous
import functools

import jax
import jax.numpy as jnp
from jax import lax
from jax.experimental import pallas as pl
from jax.experimental.pallas import tpu as pltpu
from jax.experimental.pallas import tpu_sc as plsc

EMB = 64
NC = 2
NS = 16
NW = NC * NS
CHUNK = 800


def _lookup(idx, table, b, h):
    B = idx.shape[0]
    bpw = B // NW
    rpw = b // NW
    nchunk = bpw // CHUNK
    cb = CHUNK // h

    mesh = plsc.VectorSubcoreMesh(
        core_axis_name="c", subcore_axis_name="s", num_cores=NC, num_subcores=NS
    )

    @functools.partial(
        pl.kernel,
        out_type=jax.ShapeDtypeStruct((B, EMB), jnp.float32),
        mesh=mesh,
        scratch_types=[
            pltpu.VMEM((bpw,), jnp.int32),
            pltpu.VMEM((CHUNK, EMB), jnp.float32),
            pltpu.VMEM((CHUNK, EMB), jnp.float32),
            pltpu.SemaphoreType.DMA,
            pltpu.SemaphoreType.DMA,
            pltpu.SemaphoreType.DMA,
            pltpu.SemaphoreType.DMA,
        ],
        compiler_params=pltpu.CompilerParams(use_tc_tiling_on_sc=False),
    )
    def body(idx_hbm, table_hbm, out_hbm, idx_v, rows0, rows1, g0, g1, o0, o1):
        wid = lax.axis_index("s") * NC + lax.axis_index("c")
        base = wid * bpw
        rows = (rows0, rows1)
        gsem = (g0, g1)
        osem = (o0, o1)

        pltpu.sync_copy(idx_hbm.at[pl.ds(base, bpw)], idx_v)

        def g_start(t, p):
            pltpu.async_copy(
                table_hbm.at[idx_v.at[pl.ds(t * CHUNK, CHUNK)]], rows[p], gsem[p]
            )

        def g_wait(p):
            pltpu.make_async_copy(
                table_hbm.at[idx_v.at[pl.ds(0, CHUNK)]], rows[p], gsem[p]
            ).wait()

        def o_start(t, p):
            pltpu.async_copy(
                rows[p], out_hbm.at[pl.ds(base + t * CHUNK, CHUNK)], osem[p]
            )

        def o_wait(p):
            pltpu.make_async_copy(
                rows[p], out_hbm.at[pl.ds(0, CHUNK)], osem[p]
            ).wait()

        g_start(0, 0)
        g_start(1, 1)
        g_wait(0)
        o_start(0, 0)

        @pl.loop(1, nchunk - 1, step=2)
        def mid(c):
            for p in (1, 0):
                t = c if p == 1 else c + 1
                np_ = 1 - p
                o_wait(np_)
                g_start(t + 1, np_)
                g_wait(p)
                o_start(t, p)

        g_wait(1)
        o_start(nchunk - 1, 1)
        o_wait(0)
        o_wait(1)

    return body(idx, table)


def kernel(x, emb_weight):
    b, h = x.shape
    idx = x.reshape(-1).astype(jnp.int32) * 2
    padded = jnp.pad(emb_weight, ((0, 0), (0, 64))).reshape(2000000, EMB)
    return _lookup(idx, padded, b, h).reshape(b, h, EMB)
